# Initial kernel scaffold; baseline (speedup 1.0000x reference)
import jax
import jax.numpy as jnp
from jax.experimental import pallas as pl


def kernel(h, V, U, Wr, br, W1, b1, W2, b2, Wh, bh):
    raise NotImplementedError("write your pallas kernel here")


# speedup vs baseline: 2.8295x; 2.8295x over previous
import jax
import jax.numpy as jnp
from jax.experimental import pallas as pl
from jax.experimental.pallas import tpu as pltpu

B, S, D = 1, 2048, 2048
RANK, HID = 128, 128
E, TOPK = 64, 2
BLEND = min(TOPK / E, 0.9)
T = B * S
TT = 256
NT = T // TT

_bf16 = jnp.bfloat16
_f32 = jnp.float32


def _route_kernel(h_ref, V_ref, Wr_ref, br_ref,
                  z_ref, e1_ref, e2_ref, w1_ref, w2_ref, me_ref, cnt_ref):
    ht = h_ref[...]
    z = jnp.dot(ht.astype(_bf16), V_ref[...].astype(_bf16),
                preferred_element_type=_f32)
    z_ref[...] = z
    logits = jnp.dot(z.astype(_bf16), Wr_ref[...].astype(_bf16),
                     preferred_element_type=_f32) + br_ref[...]
    m = jnp.max(logits, axis=-1, keepdims=True)
    ex = jnp.exp(logits - m)
    probs = ex / jnp.sum(ex, axis=-1, keepdims=True)
    i1 = jnp.argmax(probs, axis=-1)
    p1 = jnp.max(probs, axis=-1)
    iota_e = jax.lax.broadcasted_iota(jnp.int32, probs.shape, 1)
    masked = jnp.where(iota_e == i1[:, None], -jnp.inf, probs)
    i2 = jnp.argmax(masked, axis=-1)
    p2 = jnp.max(masked, axis=-1)
    s = p1 + p2
    e1_ref[...] = i1[:, None].astype(jnp.int32)
    e2_ref[...] = i2[:, None].astype(jnp.int32)
    w1_ref[...] = (p1 / s)[:, None]
    w2_ref[...] = (p2 / s)[:, None]
    oh = ((iota_e == i1[:, None]).astype(_f32)
          + (iota_e == i2[:, None]).astype(_f32))

    @pl.when(pl.program_id(0) == 0)
    def _():
        me_ref[...] = jnp.zeros_like(me_ref)
        cnt_ref[...] = jnp.zeros_like(cnt_ref)

    me_ref[...] += jnp.sum(probs, axis=0, keepdims=True)
    cnt_ref[...] += jnp.sum(oh, axis=0, keepdims=True)


def _experts_kernel(z_ref, e1_ref, e2_ref, w1_ref, w2_ref,
                    W1_ref, b1_ref, W2_ref, b2_ref, acc_ref):
    e = pl.program_id(0)
    zb = z_ref[...].astype(_bf16)
    x = jnp.dot(zb, W1_ref[0].astype(_bf16),
                preferred_element_type=_f32) + b1_ref[0]
    y = jax.nn.gelu(x)
    out = jnp.dot(y.astype(_bf16), W2_ref[0].astype(_bf16),
                  preferred_element_type=_f32) + b2_ref[0]
    g = (jnp.where(e1_ref[...] == e, w1_ref[...], 0.0)
         + jnp.where(e2_ref[...] == e, w2_ref[...], 0.0))

    @pl.when(e == 0)
    def _():
        acc_ref[...] = jnp.zeros_like(acc_ref)

    acc_ref[...] += g * out


def _recon_kernel(z_ref, znew_ref, U_ref, Wh_ref, bh_ref, hnew_ref, p_ref):
    zfin = znew_ref[...] * BLEND + z_ref[...] * (1.0 - BLEND)
    hn = jnp.dot(zfin.astype(_bf16), U_ref[...].astype(_bf16),
                 preferred_element_type=_f32)
    hnew_ref[...] = hn
    q = jnp.dot(hn.astype(_bf16), Wh_ref[...].astype(_bf16),
                preferred_element_type=_f32) + bh_ref[...]
    p_ref[...] = jax.nn.sigmoid(q)


def _aux_kernel(me_ref, cnt_ref, aux_ref):
    me = me_ref[...] / T
    fe = cnt_ref[...] / T
    aux_ref[...] = jnp.full((1, 1), E, _f32) * jnp.sum(me * fe)


def kernel(h, V, U, Wr, br, W1, b1, W2, b2, Wh, bh):
    hf = h.reshape(T, D)
    br2 = br.reshape(1, E)
    bh2 = bh.reshape(1, 1)

    z, e1, e2, w1, w2, me_sum, cnt = pl.pallas_call(
        _route_kernel,
        grid=(NT,),
        in_specs=[
            pl.BlockSpec((TT, D), lambda i: (i, 0)),
            pl.BlockSpec((D, RANK), lambda i: (0, 0)),
            pl.BlockSpec((RANK, E), lambda i: (0, 0)),
            pl.BlockSpec((1, E), lambda i: (0, 0)),
        ],
        out_specs=[
            pl.BlockSpec((TT, RANK), lambda i: (i, 0)),
            pl.BlockSpec((TT, 1), lambda i: (i, 0)),
            pl.BlockSpec((TT, 1), lambda i: (i, 0)),
            pl.BlockSpec((TT, 1), lambda i: (i, 0)),
            pl.BlockSpec((TT, 1), lambda i: (i, 0)),
            pl.BlockSpec((1, E), lambda i: (0, 0)),
            pl.BlockSpec((1, E), lambda i: (0, 0)),
        ],
        out_shape=[
            jax.ShapeDtypeStruct((T, RANK), _f32),
            jax.ShapeDtypeStruct((T, 1), jnp.int32),
            jax.ShapeDtypeStruct((T, 1), jnp.int32),
            jax.ShapeDtypeStruct((T, 1), _f32),
            jax.ShapeDtypeStruct((T, 1), _f32),
            jax.ShapeDtypeStruct((1, E), _f32),
            jax.ShapeDtypeStruct((1, E), _f32),
        ],
    )(hf, V, Wr, br2)

    z_new = pl.pallas_call(
        _experts_kernel,
        grid=(E,),
        in_specs=[
            pl.BlockSpec((T, RANK), lambda e: (0, 0)),
            pl.BlockSpec((T, 1), lambda e: (0, 0)),
            pl.BlockSpec((T, 1), lambda e: (0, 0)),
            pl.BlockSpec((T, 1), lambda e: (0, 0)),
            pl.BlockSpec((T, 1), lambda e: (0, 0)),
            pl.BlockSpec((1, RANK, HID), lambda e: (e, 0, 0)),
            pl.BlockSpec((1, 1, HID), lambda e: (e, 0, 0)),
            pl.BlockSpec((1, HID, RANK), lambda e: (e, 0, 0)),
            pl.BlockSpec((1, 1, RANK), lambda e: (e, 0, 0)),
        ],
        out_specs=pl.BlockSpec((T, RANK), lambda e: (0, 0)),
        out_shape=jax.ShapeDtypeStruct((T, RANK), _f32),
    )(z, e1, e2, w1, w2, W1, b1.reshape(E, 1, HID), W2, b2.reshape(E, 1, RANK))

    h_new, p = pl.pallas_call(
        _recon_kernel,
        grid=(NT,),
        in_specs=[
            pl.BlockSpec((TT, RANK), lambda i: (i, 0)),
            pl.BlockSpec((TT, RANK), lambda i: (i, 0)),
            pl.BlockSpec((RANK, D), lambda i: (0, 0)),
            pl.BlockSpec((D, 1), lambda i: (0, 0)),
            pl.BlockSpec((1, 1), lambda i: (0, 0)),
        ],
        out_specs=[
            pl.BlockSpec((TT, D), lambda i: (i, 0)),
            pl.BlockSpec((TT, 1), lambda i: (i, 0)),
        ],
        out_shape=[
            jax.ShapeDtypeStruct((T, D), _f32),
            jax.ShapeDtypeStruct((T, 1), _f32),
        ],
    )(z, z_new, U, Wh, bh2)

    aux = pl.pallas_call(
        _aux_kernel,
        in_specs=[
            pl.BlockSpec((1, E), lambda: (0, 0)),
            pl.BlockSpec((1, E), lambda: (0, 0)),
        ],
        out_specs=pl.BlockSpec((1, 1), lambda: (0, 0)),
        out_shape=jax.ShapeDtypeStruct((1, 1), _f32),
    )(me_sum, cnt)

    return (h_new.reshape(B, S, D), p.reshape(B, S), aux.reshape(()))
